# baseline (speedup 1.0000x reference)
import functools

import jax
import jax.numpy as jnp
import numpy as np
from jax import lax
from jax.experimental import pallas as pl
from jax.experimental.pallas import tpu as pltpu
from jax.experimental.pallas import tpu_sc as plsc

_K = 100
_KP = 128
_B = 512
_D = 512
_F_SCORE = 1.2447
_LN_BETA = float(np.log(0.999))
_NC, _NS, _L = 2, 16, 16
_NW = _NC * _NS
_NCH = _D // _L
_GBLK = 64

_mesh = plsc.VectorSubcoreMesh(
    core_axis_name="c", subcore_axis_name="s", num_cores=_NC, num_subcores=_NS)


def _wid():
    return lax.axis_index("c") * _NS + lax.axis_index("s")


def _lane_iota():
    return lax.broadcasted_iota(jnp.int32, (_L,), 0)


def _sdiv(a, b):
    va = jnp.zeros((_L,), jnp.float32) + a
    vb = jnp.zeros((_L,), jnp.float32) + b
    return (va / vb)[0]


def _bsum_body(logits_ref, lab_ref, out_ref):
    x = logits_ref[...]
    labv = lab_ref[...]
    m = jnp.max(x, axis=1, keepdims=True)
    e = jnp.exp(x - m)
    p = e / jnp.sum(e, axis=1, keepdims=True)
    log_p = jnp.maximum(jnp.log(p), -100.0)
    log_1mp = jnp.maximum(jnp.log(1.0 - p), -100.0)
    oh = lax.broadcasted_iota(jnp.int32, x.shape, 1) == labv
    row = (jnp.sum(jnp.where(oh, log_p - log_1mp, 0.0), axis=1, keepdims=True)
           + jnp.sum(log_1mp, axis=1, keepdims=True))
    out_ref[...] = -row


def _bsum_tc(logits, labels):
    out = pl.pallas_call(
        _bsum_body,
        out_shape=jax.ShapeDtypeStruct((_B, 1), jnp.float32),
    )(logits, labels.reshape(_B, 1))
    return out.reshape(_B)


def _pab_body(feat_hbm, lab_hbm, cnt_hbm, stats_out,
              lab_v, idxb_v, rows_v, acc4, mean4, cnt_v, stat_v, sem,
              zsum_sm, nz_sm, sz2_sm):
    w = _wid()
    lane = _lane_iota()
    pltpu.sync_copy(lab_hbm, lab_v.at[pl.ds(0, _B)])
    pltpu.sync_copy(cnt_hbm, cnt_v.at[pl.ds(0, _K)])
    zeros16 = jnp.zeros((_L,), jnp.float32)
    izeros16 = jnp.zeros((_L,), jnp.int32)

    for j in range(_B // _L + 1):
        idxb_v[pl.ds(j * _L, _L)] = izeros16

    def scan_chunk(c, cnt):
        labc = lab_v[pl.ds(c * _L, _L)]
        m = lax.rem(labc, _NW) == w
        plsc.store_compressed(idxb_v.at[pl.ds(cnt, _L)], c * _L + lane, mask=m)
        npop = plsc.all_reduce_population_count(m)
        return cnt + npop[0]
    my_n = lax.fori_loop(0, _B // _L, scan_chunk, 0)

    for r in range(4):
        for j in range(_NCH):
            acc4[r, pl.ds(j * _L, _L)] = zeros16
    nblk = lax.div(my_n + (_GBLK - 1), _GBLK)

    def blk1(g, carry):
        pltpu.async_copy(feat_hbm.at[idxb_v.at[pl.ds(g * _GBLK, _GBLK)]],
                         rows_v, sem).wait()
        lim = jnp.minimum(my_n - g * _GBLK, _GBLK)

        def row1(i, c2):
            gi = idxb_v[pl.ds(g * _GBLK + i, _L)][0]
            lab = lab_v[pl.ds(gi, _L)][0]
            r = lax.shift_right_logical(lab, 5)

            def ch(j, c3):
                acc4[r, pl.ds(j * _L, _L)] = (
                    acc4[r, pl.ds(j * _L, _L)] + rows_v[i, pl.ds(j * _L, _L)])
                return c3
            lax.fori_loop(0, _NCH, ch, 0)
            return c2
        lax.fori_loop(0, lim, row1, 0)
        return carry
    lax.fori_loop(0, nblk, blk1, 0)

    cls_idx = w + _NW * lax.rem(lane, 4)
    cnt4 = plsc.load_gather(cnt_v, [cls_idx])
    inv4 = 1.0 / cnt4
    for r in range(4):
        inv_r = inv4[r]
        for j in range(_NCH):
            mean4[r, pl.ds(j * _L, _L)] = acc4[r, pl.ds(j * _L, _L)] * inv_r

    for r in range(4):
        zsum_sm[r] = 0.0
        nz_sm[r] = 0.0
    sz2_sm[0] = 0.0

    def blk2(g, carry):
        pltpu.async_copy(feat_hbm.at[idxb_v.at[pl.ds(g * _GBLK, _GBLK)]],
                         rows_v, sem).wait()
        lim = jnp.minimum(my_n - g * _GBLK, _GBLK)

        def row2(i, c2):
            gi = idxb_v[pl.ds(g * _GBLK + i, _L)][0]
            lab = lab_v[pl.ds(gi, _L)][0]
            r = lax.shift_right_logical(lab, 5)

            def ch(j, acc):
                f = rows_v[i, pl.ds(j * _L, _L)]
                m = mean4[r, pl.ds(j * _L, _L)]
                return acc + jnp.where(f != 0.0, jnp.abs(f - m), 0.0)
            acc = lax.fori_loop(0, _NCH, ch, jnp.zeros((_L,), jnp.float32))
            z = jnp.sum(acc)
            zsum_sm[r] = zsum_sm[r] + z
            nz_sm[r] = nz_sm[r] + jnp.where(z != 0.0, 1.0, 0.0)
            sz2_sm[0] = sz2_sm[0] + z * z
            return c2
        lax.fori_loop(0, lim, row2, 0)
        return carry
    lax.fori_loop(0, nblk, blk2, 0)

    vec = jnp.zeros((_L,), jnp.float32)
    for r in range(4):
        vec = jnp.where(lane == r, zsum_sm[r], vec)
        vec = jnp.where(lane == 4 + r, nz_sm[r], vec)
    vec = jnp.where(lane == 8, sz2_sm[0], vec)
    stat_v[...] = vec
    pltpu.sync_copy(stat_v, stats_out.at[pl.ds(w * _L, _L)])


_phase_ab = functools.partial(
    pl.kernel,
    out_type=jax.ShapeDtypeStruct((_NW * _L,), jnp.float32),
    mesh=_mesh,
    compiler_params=pltpu.CompilerParams(needs_layout_passes=False),
    scratch_types=[
        pltpu.VMEM((_B + _L,), jnp.int32),
        pltpu.VMEM((_B + _L,), jnp.int32),
        pltpu.VMEM((_GBLK, _D), jnp.float32),
        pltpu.VMEM((4, _D), jnp.float32),
        pltpu.VMEM((4, _D), jnp.float32),
        pltpu.VMEM((_KP,), jnp.float32),
        pltpu.VMEM((_L,), jnp.float32),
        pltpu.SemaphoreType.DMA,
        pltpu.SMEM((4,), jnp.float32),
        pltpu.SMEM((4,), jnp.float32),
        pltpu.SMEM((1,), jnp.float32),
    ],
)(_pab_body)


def _sqrt16(x):
    xi = lax.bitcast_convert_type(x, jnp.int32)
    yi = jnp.int32(0x5F3759DF) - lax.shift_right_logical(xi, 1)
    y = lax.bitcast_convert_type(yi, jnp.float32)
    for _ in range(4):
        y = y * (1.5 - 0.5 * x * y * y)
    return x * y


def _pc_body(stats_hbm, lab_hbm, cnt_hbm, bsum_hbm, loss_out,
             stats_v, lab_v, cnt_v, bsum_v, zsum_v, zim_v, nz_v, sb_v, w_v,
             loss_v):
    @pl.when(_wid() == 0)
    def _():
        pltpu.sync_copy(stats_hbm, stats_v)
        pltpu.sync_copy(lab_hbm, lab_v)
        pltpu.sync_copy(cnt_hbm, cnt_v.at[pl.ds(0, _K)])
        pltpu.sync_copy(bsum_hbm, bsum_v)
        lane = _lane_iota()

        t0 = plsc.load_gather(stats_v, [lane * _L + 8])
        t1 = plsc.load_gather(stats_v, [(lane + _L) * _L + 8])
        sz2 = jnp.sum(t0 + t1)

        zm_acc = jnp.zeros((_L,), jnp.float32)
        n_acc = jnp.zeros((_L,), jnp.float32)
        for q in range(_KP // _L):
            cls = lane + q * _L
            tile = lax.rem(cls, _NW)
            r = lax.shift_right_logical(cls, 5)
            zsum_c = plsc.load_gather(stats_v, [tile * _L + r])
            nz_c = plsc.load_gather(stats_v, [tile * _L + 4 + r])
            valid = cls < _K
            cnt_c = jnp.where(valid, cnt_v[pl.ds(q * _L, _L)], 1.0)
            zim_c = zsum_c / cnt_c
            zsum_v[pl.ds(q * _L, _L)] = zsum_c
            zim_v[pl.ds(q * _L, _L)] = zim_c
            nz_v[pl.ds(q * _L, _L)] = nz_c
            zm_acc = zm_acc + jnp.where(valid, zim_c, 0.0)
            n_acc = n_acc + jnp.where(valid, cnt_c, 0.0)
        z_mean = jnp.sum(zm_acc) * (1.0 / _K)
        n_tot = jnp.sum(n_acc)

        cross_acc = jnp.zeros((_L,), jnp.float32)
        for q in range(_KP // _L):
            zim_c = zim_v[pl.ds(q * _L, _L)]
            zsum_c = zsum_v[pl.ds(q * _L, _L)]
            nz_c = nz_v[pl.ds(q * _L, _L)]
            cross_acc = cross_acc + zim_c * (zim_c * nz_c - 2.0 * zsum_c)
        ssw = _sdiv(sz2 + jnp.sum(cross_acc), n_tot - float(_K))

        ssb_acc = jnp.zeros((_L,), jnp.float32)
        for q in range(_KP // _L):
            valid = (_lane_iota() + q * _L) < _K
            cnt_c = jnp.where(valid, cnt_v[pl.ds(q * _L, _L)], 1.0)
            dzm = zim_v[pl.ds(q * _L, _L)] - z_mean
            sbm = jnp.where(valid, dzm * dzm * cnt_c, 0.0)
            sb_v[pl.ds(q * _L, _L)] = sbm
            ssb_acc = ssb_acc + sbm
        ssb = jnp.sum(ssb_acc) * (1.0 / (_K - 1))

        a = z_mean * z_mean
        inv2a = _sdiv(1.0, 2.0 * a)
        ws_acc = jnp.zeros((_L,), jnp.float32)
        for q in range(_KP // _L):
            valid = (_lane_iota() + q * _L) < _K
            zsum_c = zsum_v[pl.ds(q * _L, _L)]
            cnt_c = jnp.where(valid, cnt_v[pl.ds(q * _L, _L)], 1.0)
            sb_c = sb_v[pl.ds(q * _L, _L)]
            cq = _F_SCORE * ssw * float(_K - 1) - (ssb * float(_K - 1) - sb_c)
            bq = -(2.0 * z_mean * zsum_c + cq)
            d2 = bq * bq - 4.0 * a * (zsum_c * zsum_c)
            dok = d2 >= 0.0
            dq = _sqrt16(jnp.maximum(d2, 0.0))
            n_lb = jnp.abs((-bq - dq) * inv2a)
            n_ub = jnp.abs((-bq + dq) * inv2a)
            c1 = jnp.logical_and(dok, cnt_c < n_lb)
            c2 = jnp.logical_and(dok, cnt_c > n_ub)
            t = jnp.where(c1, 1.0 / (n_lb - cnt_c),
                          jnp.where(c2, 1.0 / (cnt_c - n_ub), 1.0))
            beta = jnp.exp(_LN_BETA * t)
            en = 1.0 - jnp.exp(_LN_BETA * t * cnt_c)
            wr = (1.0 - beta) / en
            wrm = jnp.where(valid, wr, 0.0)
            w_v[pl.ds(q * _L, _L)] = wrm
            ws_acc = ws_acc + wrm
        wsum = jnp.sum(ws_acc)

        def dotc(c, acc):
            labc = lab_v[pl.ds(c * _L, _L)]
            wg = plsc.load_gather(w_v, [labc])
            return acc + wg * bsum_v[pl.ds(c * _L, _L)]
        dot_acc = lax.fori_loop(0, _B // _L, dotc,
                                jnp.zeros((_L,), jnp.float32))
        loss = jnp.sum(dot_acc) * _sdiv(float(_K), wsum) * (1.0 / (_B * _K))
        loss_v[...] = jnp.zeros((_L,), jnp.float32) + loss
        pltpu.sync_copy(loss_v, loss_out)


_phase_c = functools.partial(
    pl.kernel,
    out_type=jax.ShapeDtypeStruct((_L,), jnp.float32),
    mesh=_mesh,
    compiler_params=pltpu.CompilerParams(needs_layout_passes=False),
    scratch_types=[
        pltpu.VMEM((_NW * _L,), jnp.float32),
        pltpu.VMEM((_B,), jnp.int32),
        pltpu.VMEM((_KP,), jnp.float32),
        pltpu.VMEM((_B,), jnp.float32),
        pltpu.VMEM((_KP,), jnp.float32),
        pltpu.VMEM((_KP,), jnp.float32),
        pltpu.VMEM((_KP,), jnp.float32),
        pltpu.VMEM((_KP,), jnp.float32),
        pltpu.VMEM((_KP,), jnp.float32),
        pltpu.VMEM((_L,), jnp.float32),
    ],
)(_pc_body)


def kernel(logits, labels, features, sample_num_per_cls):
    labels = labels.astype(jnp.int32)
    bsum = _bsum_tc(logits, labels)
    stats = _phase_ab(features, labels, sample_num_per_cls)
    loss_vec = _phase_c(stats, labels, sample_num_per_cls, bsum)
    return loss_vec[0]
